# baseline (speedup 1.0000x reference)
import functools

import jax
import jax.numpy as jnp
from jax import lax
from jax.experimental import pallas as pl
from jax.experimental.pallas import tpu as pltpu
from jax.experimental.pallas import tpu_sc as plsc

_Q_NUM = 100000
_TIME_SPEND = 1000
_SEQ_LEN = 200
_D = 64
_W = _D // 2
_BATCH = 4096
_N = _BATCH * _SEQ_LEN

_info = plsc.get_sparse_core_info()
_NC, _NS = _info.num_cores, _info.num_subcores
_NW = _NC * _NS
_TPW = _N // _NW
_C = 128
_NCH = _TPW // _C
_TOTCH = _N // _C

_mesh = plsc.VectorSubcoreMesh(core_axis_name="c", subcore_axis_name="s")


@functools.partial(
    pl.kernel,
    out_type=jax.ShapeDtypeStruct((_N, _D), jnp.float32),
    mesh=_mesh,
    compiler_params=pltpu.CompilerParams(use_tc_tiling_on_sc=False,
                                         needs_layout_passes=False),
    scratch_types=[
        pltpu.VMEM((_NCH, 2, _C), jnp.int32),
        pltpu.VMEM((_TIME_SPEND * _W,), jnp.int32),
        pltpu.VMEM((80 * _W,), jnp.int32),
        pltpu.VMEM((_SEQ_LEN * _W,), jnp.int32),
        pltpu.VMEM((_C, _W), jnp.int32),
        pltpu.VMEM((_C, _W), jnp.int32),
        pltpu.VMEM((_C, _D), jnp.float32),
        pltpu.VMEM((_C, _D), jnp.float32),
        pltpu.SemaphoreType.DMA,
        pltpu.SemaphoreType.DMA,
        pltpu.SemaphoreType.DMA,
        pltpu.SemaphoreType.DMA,
    ],
)
def _sc_embed(meta_h, Wex_h, Wcat_h, Wrs_h, Wpos_h, out_h,
              idxall, catv, rsv, posv,
              exb0, exb1, outb0, outb1, sem0, sem1, wsem0, wsem1):
    wid = lax.axis_index("s") * _NC + lax.axis_index("c")
    cgbase = wid * _NCH

    exb = (exb0, exb1)
    outbs = (outb0, outb1)
    sems = (sem0, sem1)
    wsems = (wsem0, wsem1)

    pltpu.sync_copy(Wcat_h, catv)
    pltpu.sync_copy(Wrs_h, rsv)
    pltpu.sync_copy(Wpos_h, posv)
    pltpu.sync_copy(meta_h.at[wid], idxall)

    cols = [lax.iota(jnp.int32, 16) + 16 * q for q in range(2)]
    _dnums = lax.GatherDimensionNumbers(
        offset_dims=(), collapsed_slice_dims=(0,), start_index_map=(0,))
    lane_consts = [jnp.full((16, 1), j, jnp.int32) for j in range(16)]
    himask = jnp.int32(-65536)

    def lane_bcast(vec, j):
        return lax.gather(vec, lane_consts[j], _dnums, (1,),
                          mode=lax.GatherScatterMode.PROMISE_IN_BOUNDS)

    def as_bf(w):
        return plsc.bitcast(w, jnp.bfloat16)

    def fire(g, b):
        pltpu.async_copy(Wex_h.at[idxall.at[g, 0]], exb[b], sems[b])

    fire(0, 0)
    fire(1, 1)

    def outer(i, carry):
        for b in (0, 1):
            g = i * 2 + b

            pltpu.make_async_copy(Wex_h.at[idxall.at[0, 0]], exb[b],
                                  sems[b]).wait()
            outb = outbs[b]

            @pl.when(g >= 2)
            def _():
                pltpu.make_async_copy(
                    outb, out_h.at[pl.ds((cgbase + g - 2) * _C, _C)],
                    wsems[b]).wait()

            @plsc.parallel_loop(0, _C // 16, unroll=2)
            def group(m):
                svec = idxall[g, 1, pl.ds(16 * m, 16)]
                cg_ = (svec >> 15) << 5
                rg_ = ((svec >> 8) & 127) << 5
                pg_ = (svec & 255) << 5
                for j in range(16):
                    t = m * 16 + j
                    cvec = lane_bcast(cg_, j)
                    rvec = lane_bcast(rg_, j)
                    pvec = lane_bcast(pg_, j)
                    for q in range(2):
                        ex_q = as_bf(exb[b][t, pl.ds(16 * q, 16)])
                        c_q = as_bf(plsc.load_gather(catv, [cvec + cols[q]]))
                        r_q = as_bf(plsc.load_gather(rsv, [rvec + cols[q]]))
                        p_q = as_bf(plsc.load_gather(posv, [pvec + cols[q]]))
                        s = (ex_q + c_q) + (r_q + p_q)
                        sw = plsc.bitcast(s, jnp.int32)
                        outb[t, pl.ds(32 * q, 16)] = plsc.bitcast(
                            lax.shift_left(sw, 16), jnp.float32)
                        outb[t, pl.ds(32 * q + 16, 16)] = plsc.bitcast(
                            lax.bitwise_and(sw, himask), jnp.float32)

            @pl.when(g + 2 < _NCH)
            def _():
                fire(g + 2, b)

            pltpu.async_copy(outb, out_h.at[pl.ds((cgbase + g) * _C, _C)],
                             wsems[b])
        return carry

    lax.fori_loop(0, _NCH // 2, outer, 0)

    for b in (0, 1):
        g = _NCH - 2 + b
        pltpu.make_async_copy(
            outbs[b], out_h.at[pl.ds((cgbase + g) * _C, _C)],
            wsems[b]).wait()


def _pack_bf16(tab):
    u = lax.bitcast_convert_type(tab.astype(jnp.bfloat16),
                                 jnp.uint16).astype(jnp.uint32)
    w = jnp.concatenate([u[:, 0:16] | (u[:, 16:32] << 16),
                         u[:, 32:48] | (u[:, 48:64] << 16)], axis=1)
    return lax.bitcast_convert_type(w, jnp.int32)


def kernel(exercises, categories, response, skill, W_ex, W_cat, W_pos,
           W_resp, W_skill):
    ex = exercises.reshape(-1).astype(jnp.int32)
    cat = categories.reshape(-1).astype(jnp.int32)
    rs = (response * 40 + skill).reshape(-1).astype(jnp.int32)
    pos = jnp.broadcast_to(
        jnp.arange(_SEQ_LEN, dtype=jnp.int32)[None, :],
        (_BATCH, _SEQ_LEN)).reshape(-1)
    packed = (cat << 15) | (rs << 8) | pos
    meta = jnp.stack([ex.reshape(_NW, _NCH, _C),
                      packed.reshape(_NW, _NCH, _C)], axis=2)
    W_rs = (W_resp[:, None, :] + W_skill[None, :, :]).reshape(80, _D)
    out = _sc_embed(meta, _pack_bf16(W_ex), _pack_bf16(W_cat).reshape(-1),
                    _pack_bf16(W_rs).reshape(-1),
                    _pack_bf16(W_pos).reshape(-1))
    return out.reshape(_BATCH, _SEQ_LEN, _D)


# speedup vs baseline: 1.3228x; 1.3228x over previous
import functools

import jax
import jax.numpy as jnp
from jax import lax
from jax.experimental import pallas as pl
from jax.experimental.pallas import tpu as pltpu
from jax.experimental.pallas import tpu_sc as plsc

_Q_NUM = 100000
_TIME_SPEND = 1000
_SEQ_LEN = 200
_D = 64
_W = _D // 2
_BATCH = 4096
_N = _BATCH * _SEQ_LEN

_info = plsc.get_sparse_core_info()
_NC, _NS = _info.num_cores, _info.num_subcores
_NW = _NC * _NS
_TPW = _N // _NW
_C = 128
_NCH = _TPW // _C
_TOTCH = _N // _C

_mesh = plsc.VectorSubcoreMesh(core_axis_name="c", subcore_axis_name="s")


@functools.partial(
    pl.kernel,
    out_type=jax.ShapeDtypeStruct((_N, _D), jnp.float32),
    mesh=_mesh,
    compiler_params=pltpu.CompilerParams(use_tc_tiling_on_sc=False,
                                         needs_layout_passes=False),
    scratch_types=[
        pltpu.VMEM((_NCH, 2, _C), jnp.int32),
        pltpu.VMEM((_TIME_SPEND, _W), jnp.int32),
        pltpu.VMEM((80, _W), jnp.int32),
        pltpu.VMEM((_SEQ_LEN, _W), jnp.int32),
        pltpu.VMEM((_C, _W), jnp.int32),
        pltpu.VMEM((_C, _W), jnp.int32),
        pltpu.VMEM((_C, _D), jnp.float32),
        pltpu.VMEM((_C, _D), jnp.float32),
        pltpu.SemaphoreType.DMA,
        pltpu.SemaphoreType.DMA,
        pltpu.SemaphoreType.DMA,
        pltpu.SemaphoreType.DMA,
    ],
)
def _sc_embed(meta_h, Wex_h, Wcat_h, Wrs_h, Wpos_h, out_h,
              idxall, catv, rsv, posv,
              exb0, exb1, outb0, outb1, sem0, sem1, wsem0, wsem1):
    wid = lax.axis_index("s") * _NC + lax.axis_index("c")
    cgbase = wid * _NCH

    exb = (exb0, exb1)
    outbs = (outb0, outb1)
    sems = (sem0, sem1)
    wsems = (wsem0, wsem1)

    pltpu.sync_copy(Wcat_h, catv)
    pltpu.sync_copy(Wrs_h, rsv)
    pltpu.sync_copy(Wpos_h, posv)
    pltpu.sync_copy(meta_h.at[wid], idxall)

    cols = [lax.iota(jnp.int32, 16) + 16 * q for q in range(2)]
    _dnums = lax.GatherDimensionNumbers(
        offset_dims=(), collapsed_slice_dims=(0,), start_index_map=(0,))
    lane_consts = [jnp.full((16, 1), j, jnp.int32) for j in range(16)]
    himask = jnp.int32(-65536)

    def lane_bcast(vec, j):
        return lax.gather(vec, lane_consts[j], _dnums, (1,),
                          mode=lax.GatherScatterMode.PROMISE_IN_BOUNDS)

    def as_bf(w):
        return plsc.bitcast(w, jnp.bfloat16)

    def fire(g, b):
        pltpu.async_copy(Wex_h.at[idxall.at[g, 0]], exb[b], sems[b])

    fire(0, 0)
    fire(1, 1)

    def outer(i, carry):
        for b in (0, 1):
            g = i * 2 + b

            pltpu.make_async_copy(Wex_h.at[idxall.at[0, 0]], exb[b],
                                  sems[b]).wait()
            outb = outbs[b]

            @pl.when(g >= 2)
            def _():
                pltpu.make_async_copy(
                    outb, out_h.at[pl.ds((cgbase + g - 2) * _C, _C)],
                    wsems[b]).wait()

            @plsc.parallel_loop(0, _C // 16, unroll=2)
            def group(m):
                svec = idxall[g, 1, pl.ds(16 * m, 16)]
                cg_ = svec >> 15
                rg_ = (svec >> 8) & 127
                pg_ = svec & 255
                for j in range(16):
                    t = m * 16 + j
                    cvec = lane_bcast(cg_, j)
                    rvec = lane_bcast(rg_, j)
                    pvec = lane_bcast(pg_, j)
                    for q in range(2):
                        ex_q = as_bf(exb[b][t, pl.ds(16 * q, 16)])
                        c_q = as_bf(plsc.load_gather(catv, [cvec, cols[q]]))
                        r_q = as_bf(plsc.load_gather(rsv, [rvec, cols[q]]))
                        p_q = as_bf(plsc.load_gather(posv, [pvec, cols[q]]))
                        s = (ex_q + c_q) + (r_q + p_q)
                        sw = plsc.bitcast(s, jnp.int32)
                        outb[t, pl.ds(32 * q, 16)] = plsc.bitcast(
                            lax.shift_left(sw, 16), jnp.float32)
                        outb[t, pl.ds(32 * q + 16, 16)] = plsc.bitcast(
                            lax.bitwise_and(sw, himask), jnp.float32)

            @pl.when(g + 2 < _NCH)
            def _():
                fire(g + 2, b)

            pltpu.async_copy(outb, out_h.at[pl.ds((cgbase + g) * _C, _C)],
                             wsems[b])
        return carry

    lax.fori_loop(0, _NCH // 2, outer, 0)

    for b in (0, 1):
        g = _NCH - 2 + b
        pltpu.make_async_copy(
            outbs[b], out_h.at[pl.ds((cgbase + g) * _C, _C)],
            wsems[b]).wait()


def _pack_bf16(tab):
    u = lax.bitcast_convert_type(tab.astype(jnp.bfloat16),
                                 jnp.uint16).astype(jnp.uint32)
    w = jnp.concatenate([u[:, 0:16] | (u[:, 16:32] << 16),
                         u[:, 32:48] | (u[:, 48:64] << 16)], axis=1)
    return lax.bitcast_convert_type(w, jnp.int32)


def kernel(exercises, categories, response, skill, W_ex, W_cat, W_pos,
           W_resp, W_skill):
    ex = exercises.reshape(-1).astype(jnp.int32)
    cat = categories.reshape(-1).astype(jnp.int32)
    rs = (response * 40 + skill).reshape(-1).astype(jnp.int32)
    pos = jnp.broadcast_to(
        jnp.arange(_SEQ_LEN, dtype=jnp.int32)[None, :],
        (_BATCH, _SEQ_LEN)).reshape(-1)
    packed = (cat << 15) | (rs << 8) | pos
    meta = jnp.stack([ex.reshape(_NW, _NCH, _C),
                      packed.reshape(_NW, _NCH, _C)], axis=2)
    W_rs = (W_resp[:, None, :] + W_skill[None, :, :]).reshape(80, _D)
    out = _sc_embed(meta, _pack_bf16(W_ex), _pack_bf16(W_cat),
                    _pack_bf16(W_rs), _pack_bf16(W_pos))
    return out.reshape(_BATCH, _SEQ_LEN, _D)
